# baseline (speedup 1.0000x reference)
import functools

import jax
import jax.numpy as jnp
from jax import lax
from jax.experimental import pallas as pl
from jax.experimental.pallas import tpu as pltpu
from jax.experimental.pallas import tpu_sc as plsc

B = 128
V = 100000
C = 4096
NSEG = 10
QUOTA = 408
CP = NSEG * QUOTA
W = 8192
NT = 13
RUNGS = (5.0, 6.0, 7.0, 7.5, 8.0, 8.25, 8.5, 8.75, 9.0, 10.0)
NR = len(RUNGS)
CNT_RUNGS = (6, 8, 9)
CNT_BOUND = (0, 0, 0, 0, 0, 0, 0, 1, 1, 2)
TOPP = 0.9
MASKSL = 3200
NIT = 90000


def _fold_max(x):
    acc = x[:, 0:128]
    for gi in range(1, x.shape[1] // 128):
        acc = jnp.maximum(acc, x[:, gi * 128:(gi + 1) * 128])
    return acc


def _fold_sum(x):
    acc = x[:, 0:128]
    for gi in range(1, x.shape[1] // 128):
        acc = acc + x[:, gi * 128:(gi + 1) * 128]
    return acc


def _sc_mask_body(nit_hbm, mask_hbm, idxb, mb, mbz):
    info = plsc.get_sparse_core_info()
    nc = info.num_cores
    wid = lax.axis_index("s") * nc + lax.axis_index("c")
    lo = wid * MASKSL
    pltpu.sync_copy(nit_hbm, idxb)

    def zbody(i, _):
        mb[pl.ds(i * 16, 16)] = jnp.zeros((16,), jnp.float32)
        return 0

    lax.fori_loop(0, MASKSL // 16, zbody, 0)

    ones = jnp.ones((16,), jnp.float32)

    def sbody(i, _):
        iv = idxb[pl.ds(i * 16, 16)]
        ok = (iv >= lo) & (iv < lo + MASKSL)
        plsc.store_scatter(mb, [iv - lo], ones, mask=ok)
        return 0

    lax.fori_loop(0, NIT // 16, sbody, 0, unroll=8)

    @pl.when(wid < 31)
    def _():
        pltpu.sync_copy(mb, mask_hbm.at[pl.ds(lo, MASKSL)])

    @pl.when(wid == 31)
    def _():
        pltpu.sync_copy(mb.at[pl.ds(0, V - 31 * MASKSL)],
                        mask_hbm.at[pl.ds(31 * MASKSL, V - 31 * MASKSL)])


def _sc_mask(nit):
    mesh = plsc.VectorSubcoreMesh(core_axis_name="c", subcore_axis_name="s")
    f = pl.kernel(
        _sc_mask_body,
        out_type=jax.ShapeDtypeStruct((V,), jnp.float32),
        mesh=mesh,
        compiler_params=pltpu.CompilerParams(use_tc_tiling_on_sc=False, needs_layout_passes=False),
        scratch_types=[
            pltpu.VMEM((NIT,), jnp.int32),
            pltpu.VMEM((MASKSL,), jnp.float32),
            pltpu.VMEM((16,), jnp.float32),
        ],
    )
    return f(nit)


def _a1_body(c_ref, u_ref, mk_ref, mix_ref, m_ref, macc):
    j = pl.program_id(0)
    cv = c_ref[...]
    uv = u_ref[...]
    mk = mk_ref[...]
    x = uv + 3.0 * (cv - uv)
    x = jnp.where(mk > 0.0, -100.0, x)
    col = j * W + lax.broadcasted_iota(jnp.int32, (B, W), 1)
    x = jnp.where(col < V, x, -jnp.inf)
    mix_ref[...] = x
    xm = _fold_max(x)

    @pl.when(j == 0)
    def _():
        macc[...] = xm

    @pl.when(j > 0)
    def _():
        macc[...] = jnp.maximum(macc[...], xm)

    @pl.when(j == NT - 1)
    def _():
        m_ref[...] = macc[...].max(axis=1, keepdims=True)


def _a1(cond, uncond, mask):
    return pl.pallas_call(
        _a1_body,
        grid=(NT,),
        in_specs=[
            pl.BlockSpec((B, W), lambda j: (0, j)),
            pl.BlockSpec((B, W), lambda j: (0, j)),
            pl.BlockSpec((1, W), lambda j: (0, j)),
        ],
        out_specs=[
            pl.BlockSpec((B, W), lambda j: (0, j)),
            pl.BlockSpec((B, 1), lambda j: (0, 0)),
        ],
        out_shape=[
            jax.ShapeDtypeStruct((B, V), jnp.float32),
            jax.ShapeDtypeStruct((B, 1), jnp.float32),
        ],
        scratch_shapes=[pltpu.VMEM((B, 128), jnp.float32)],
    )(cond, uncond, mask)


def _a2_body(mix_ref, m_ref, mass_ref, cnt_ref, s_ref, massacc, cntacc, sacc):
    j = pl.program_id(0)
    x = mix_ref[...]
    m = m_ref[...]
    col = j * W + lax.broadcasted_iota(jnp.int32, (B, W), 1)
    valid = col < V
    x = jnp.where(valid, x, -jnp.inf)
    e = jnp.exp(x - m)
    colok = valid & (col != 1) & (col != 2)
    e_ok = jnp.where(colok, e, 0.0)

    se = _fold_sum(e)

    @pl.when(j == 0)
    def _():
        sacc[...] = se
        massacc[...] = jnp.zeros_like(massacc)
        cntacc[...] = jnp.zeros_like(cntacc)

    @pl.when(j > 0)
    def _():
        sacc[...] = sacc[...] + se

    for r in range(NR):
        ge = x >= (m - RUNGS[r])
        mr = _fold_sum(jnp.where(ge, e_ok, 0.0))
        massacc[r] = massacc[r] + mr
    for ci, r in enumerate(CNT_RUNGS):
        ge = x >= (m - RUNGS[r])
        cr = _fold_sum(jnp.where(ge & colok, 1.0, 0.0))
        cntacc[ci] = cntacc[ci] + cr

    @pl.when(j == NT - 1)
    def _():
        for r in range(NR):
            mass_ref[:, r:r + 1] = massacc[r].sum(axis=1, keepdims=True)
        for ci in range(len(CNT_RUNGS)):
            cnt_ref[:, ci:ci + 1] = cntacc[ci].sum(axis=1, keepdims=True)
        s_ref[...] = sacc[...].sum(axis=1, keepdims=True)


def _a2(mixed, m):
    return pl.pallas_call(
        _a2_body,
        grid=(NT,),
        in_specs=[
            pl.BlockSpec((B, W), lambda j: (0, j)),
            pl.BlockSpec((B, 1), lambda j: (0, 0)),
        ],
        out_specs=[
            pl.BlockSpec((B, 128), lambda j: (0, 0)),
            pl.BlockSpec((B, 128), lambda j: (0, 0)),
            pl.BlockSpec((B, 1), lambda j: (0, 0)),
        ],
        out_shape=[
            jax.ShapeDtypeStruct((B, 128), jnp.float32),
            jax.ShapeDtypeStruct((B, 128), jnp.float32),
            jax.ShapeDtypeStruct((B, 1), jnp.float32),
        ],
        scratch_shapes=[
            pltpu.VMEM((NR, B, 128), jnp.float32),
            pltpu.VMEM((len(CNT_RUNGS), B, 128), jnp.float32),
            pltpu.VMEM((B, 128), jnp.float32),
        ],
    )(mixed, m)


def _sc_compact_body(mix_hbm, tau_hbm, cv_hbm, ci_hbm, cnt_hbm,
                     rowbuf, vbuf, ibuf, tbuf, cbuf):
    info = plsc.get_sparse_core_info()
    nc = info.num_cores
    wid = lax.axis_index("s") * nc + lax.axis_index("c")
    segv = V // NSEG
    for k in range(B // 32):
        r = wid * (B // 32) + k
        pltpu.sync_copy(mix_hbm.at[r], rowbuf)
        pltpu.sync_copy(tau_hbm.at[r], tbuf)
        tauv = tbuf[...]

        def body_fn(i, offs):
            iv0 = lax.iota(jnp.int32, 16) + i * 16
            new = []
            for s in range(NSEG):
                xv = rowbuf[pl.ds(s * segv + i * 16, 16)]
                ok = xv >= tauv
                if s == 0:
                    idv = iv0
                    ok = ok & ((idv < 1) | (idv > 2))
                else:
                    idv = iv0 + (s * segv)
                ok = ok & (offs[s] < QUOTA - 15)
                base = s * QUOTA
                plsc.store_compressed(vbuf.at[pl.ds(base + offs[s], 16)],
                                      xv, mask=ok)
                plsc.store_compressed(ibuf.at[pl.ds(base + offs[s], 16)],
                                      idv, mask=ok)
                n = plsc.all_reduce_population_count(ok)[0]
                new.append(offs[s] + n)
            return tuple(new)

        offs = lax.fori_loop(0, segv // 16, body_fn,
                             tuple(jnp.int32(0) for _ in range(NSEG)),
                             unroll=4)
        lanes = lax.iota(jnp.int32, 16)
        cvec = jnp.zeros((16,), jnp.int32)
        for s in range(NSEG):
            cvec = jnp.where(lanes == s, offs[s], cvec)
        cbuf[...] = cvec
        pltpu.sync_copy(vbuf, cv_hbm.at[r])
        pltpu.sync_copy(ibuf, ci_hbm.at[r])
        pltpu.sync_copy(cbuf, cnt_hbm.at[r])


def _sc_compact(mixed, tau16):
    mesh = plsc.VectorSubcoreMesh(core_axis_name="c", subcore_axis_name="s")
    f = pl.kernel(
        _sc_compact_body,
        out_type=[
            jax.ShapeDtypeStruct((B, CP), jnp.float32),
            jax.ShapeDtypeStruct((B, CP), jnp.int32),
            jax.ShapeDtypeStruct((B, 16), jnp.int32),
        ],
        mesh=mesh,
        compiler_params=pltpu.CompilerParams(use_tc_tiling_on_sc=False, needs_layout_passes=False),
        scratch_types=[
            pltpu.VMEM((V,), jnp.float32),
            pltpu.VMEM((CP,), jnp.float32),
            pltpu.VMEM((CP,), jnp.int32),
            pltpu.VMEM((16,), jnp.float32),
            pltpu.VMEM((16,), jnp.int32),
        ],
    )
    return f(mixed, tau16)


def _rol(x, sh):
    return jnp.concatenate([x[:, sh:], x[:, :sh]], axis=1)


def _ror(x, sh):
    return jnp.concatenate([x[:, -sh:], x[:, :-sh]], axis=1)


def _c_body(p_ref, i_ref, g_ref, tok_ref, pwin_ref):
    pv = p_ref[...]
    iv = i_ref[...]
    br = pv.shape[0]
    cols = lax.broadcasted_iota(jnp.int32, (br, C), 1)
    k = 2
    while k <= C:
        indesc = (cols & k) == 0
        j = k // 2
        while j >= 1:
            lower = (cols & j) == 0
            pp = jnp.where(lower, _rol(pv, j), _ror(pv, j))
            ip = jnp.where(lower, _rol(iv, j), _ror(iv, j))
            take_max = indesc == lower
            plarger = (pp > pv) | ((pp == pv) & (ip < iv))
            takep = take_max == plarger
            pv = jnp.where(takep, pp, pv)
            iv = jnp.where(takep, ip, iv)
            j //= 2
        k *= 2
    sp = jnp.where(pv >= 0.0, pv, 0.0)
    incl = sp
    sh = 1
    while sh < C:
        z = jnp.zeros((br, sh), jnp.float32)
        incl = incl + jnp.concatenate([z, incl[:, :C - sh]], axis=1)
        sh *= 2
    kept = ((incl - sp) < TOPP) & (pv >= 0.0)
    val = jnp.where(kept, jnp.log(jnp.maximum(pv, 1e-37)) + g_ref[...], -1e30)
    w = jnp.argmax(val, axis=-1).astype(jnp.int32)
    hot = cols == w[:, None]
    tok_ref[...] = jnp.sum(jnp.where(hot, iv, 0), axis=1, keepdims=True)
    pwin_ref[...] = jnp.sum(jnp.where(hot, sp, 0.0), axis=1, keepdims=True)


def _c_sample(p, idxs, g):
    br = B // 4
    return pl.pallas_call(
        _c_body,
        grid=(4,),
        in_specs=[
            pl.BlockSpec((br, C), lambda i: (i, 0)),
            pl.BlockSpec((br, C), lambda i: (i, 0)),
            pl.BlockSpec((br, C), lambda i: (i, 0)),
        ],
        out_specs=[
            pl.BlockSpec((br, 1), lambda i: (i, 0)),
            pl.BlockSpec((br, 1), lambda i: (i, 0)),
        ],
        out_shape=[
            jax.ShapeDtypeStruct((B, 1), jnp.int32),
            jax.ShapeDtypeStruct((B, 1), jnp.float32),
        ],
    )(p, idxs, g)


def _threefry2x32(k1, k2, x0, x1):
    rot = ((13, 15, 26, 6), (17, 29, 16, 24))

    def rotl(x, d):
        return (x << jnp.uint32(d)) | (x >> jnp.uint32(32 - d))

    ks = (k1, k2, k1 ^ k2 ^ jnp.uint32(0x1BD11BDA))
    x0 = x0 + ks[0]
    x1 = x1 + ks[1]
    for i in range(5):
        for r in rot[i % 2]:
            x0 = x0 + x1
            x1 = rotl(x1, r)
            x1 = x1 ^ x0
        x0 = x0 + ks[(i + 1) % 3]
        x1 = x1 + ks[(i + 2) % 3] + jnp.uint32(i + 1)
    return x0, x1


def _partial_gumbel(kd):
    b = lax.broadcasted_iota(jnp.uint32, (B, C), 0)
    j = lax.broadcasted_iota(jnp.uint32, (B, C), 1)
    c_lo = b * jnp.uint32(V) + j
    c_hi = jnp.zeros((B, C), jnp.uint32)
    o1, o2 = _threefry2x32(kd[0], kd[1], c_hi, c_lo)
    bits = o1 ^ o2
    fb = (bits >> jnp.uint32(9)) | jnp.uint32(0x3F800000)
    floats = lax.bitcast_convert_type(fb, jnp.float32) - jnp.float32(1.0)
    tiny = jnp.float32(jnp.finfo(jnp.float32).tiny)
    u = jnp.maximum(tiny, floats * jnp.float32(1.0) + tiny)
    return -jnp.log(-jnp.log(u))


@jax.jit
def kernel(conditioned_logits, unconditioned_logits, non_image_tokens):
    nit = non_image_tokens.astype(jnp.int32)
    mask = _sc_mask(nit)
    mixed, m = _a1(conditioned_logits, unconditioned_logits,
                   mask.reshape(1, V))
    mass_o, cnt_o, s_o = _a2(mixed, m)
    mass = mass_o[:, :NR]
    cnt = cnt_o[:, :NR][:, jnp.asarray(CNT_BOUND)]
    s = s_o[:, 0]
    rungs = jnp.asarray(RUNGS, jnp.float32)
    need = jnp.float32(TOPP) * s * jnp.float32(1.00001)
    ok = (mass >= need[:, None]) & (cnt <= CP)
    fits = cnt <= CP
    fallback = jnp.where(fits.any(axis=1),
                         NR - 1 - jnp.argmax(fits[:, ::-1], axis=1), 0)
    rung_idx = jnp.where(ok.any(axis=1), jnp.argmax(ok, axis=1), fallback)
    tau = m[:, 0] - rungs[rung_idx]
    tau16 = jnp.broadcast_to(tau[:, None], (B, 16))

    cand_v, cand_i, cnt16 = _sc_compact(mixed, tau16)
    pos = jnp.arange(CP, dtype=jnp.int32)
    segidx = pos // QUOTA
    seg_cnt = cnt16[:, segidx]
    validc = (pos % QUOTA)[None, :] < seg_cnt
    logs = jnp.log(s)
    lp = (cand_v - m) - logs[:, None]
    p = jnp.where(validc, jnp.exp(lp), jnp.float32(-1.0))
    idxs = jnp.where(validc, cand_i, jnp.int32(2 ** 30))
    p = jnp.concatenate(
        [p, jnp.full((B, C - CP), -1.0, jnp.float32)], axis=1)
    idxs = jnp.concatenate(
        [idxs, jnp.full((B, C - CP), 2 ** 30, jnp.int32)], axis=1)

    kd = jax.random.key_data(jax.random.fold_in(jax.random.key(0), 123))
    g = _partial_gumbel(kd.astype(jnp.uint32))

    tok, pwin = _c_sample(p, idxs, g)
    next_scores = jnp.log(pwin[:, 0])
    next_toks = tok[:, 0]
    return next_scores, next_toks


# speedup vs baseline: 1.0683x; 1.0683x over previous
import functools

import jax
import jax.numpy as jnp
from jax import lax
from jax.experimental import pallas as pl
from jax.experimental.pallas import tpu as pltpu
from jax.experimental.pallas import tpu_sc as plsc

B = 128
V = 100000
C = 4096
NSEG = 10
QUOTA = 408
CP = NSEG * QUOTA
W = 8192
NT = 13
RUNGS = (5.0, 6.0, 7.0, 7.5, 8.0, 8.25, 8.5, 8.75, 9.0, 10.0)
NR = len(RUNGS)
CNT_RUNGS = (6, 8, 9)
CNT_BOUND = (0, 0, 0, 0, 0, 0, 0, 1, 1, 2)
TOPP = 0.9
MASKSL = 3200
NIT = 90000


def _fold_max(x):
    acc = x[:, 0:128]
    for gi in range(1, x.shape[1] // 128):
        acc = jnp.maximum(acc, x[:, gi * 128:(gi + 1) * 128])
    return acc


def _fold_sum(x):
    acc = x[:, 0:128]
    for gi in range(1, x.shape[1] // 128):
        acc = acc + x[:, gi * 128:(gi + 1) * 128]
    return acc


def _sc_mask_body(nit_hbm, mask_hbm, idxb, mb, mbz):
    info = plsc.get_sparse_core_info()
    nc = info.num_cores
    wid = lax.axis_index("s") * nc + lax.axis_index("c")
    lo = wid * MASKSL
    pltpu.sync_copy(nit_hbm, idxb)

    def zbody(i, _):
        mb[pl.ds(i * 16, 16)] = jnp.zeros((16,), jnp.float32)
        return 0

    lax.fori_loop(0, MASKSL // 16, zbody, 0)

    ones = jnp.ones((16,), jnp.float32)

    def sbody(i, _):
        iv = idxb[pl.ds(i * 16, 16)]
        ok = (iv >= lo) & (iv < lo + MASKSL)
        plsc.store_scatter(mb, [iv - lo], ones, mask=ok)
        return 0

    lax.fori_loop(0, NIT // 16, sbody, 0, unroll=8)

    @pl.when(wid < 31)
    def _():
        pltpu.sync_copy(mb, mask_hbm.at[pl.ds(lo, MASKSL)])

    @pl.when(wid == 31)
    def _():
        pltpu.sync_copy(mb.at[pl.ds(0, V - 31 * MASKSL)],
                        mask_hbm.at[pl.ds(31 * MASKSL, V - 31 * MASKSL)])


def _sc_mask(nit):
    mesh = plsc.VectorSubcoreMesh(core_axis_name="c", subcore_axis_name="s")
    f = pl.kernel(
        _sc_mask_body,
        out_type=jax.ShapeDtypeStruct((V,), jnp.float32),
        mesh=mesh,
        compiler_params=pltpu.CompilerParams(use_tc_tiling_on_sc=False, needs_layout_passes=False),
        scratch_types=[
            pltpu.VMEM((NIT,), jnp.int32),
            pltpu.VMEM((MASKSL,), jnp.float32),
            pltpu.VMEM((16,), jnp.float32),
        ],
    )
    return f(nit)


def _a1_body(c_ref, u_ref, mk_ref, mix_ref, m_ref, macc):
    j = pl.program_id(0)
    cv = c_ref[...]
    uv = u_ref[...]
    mk = mk_ref[...]
    x = uv + 3.0 * (cv - uv)
    x = jnp.where(mk > 0.0, -100.0, x)
    col = j * W + lax.broadcasted_iota(jnp.int32, (B, W), 1)
    x = jnp.where(col < V, x, -jnp.inf)
    mix_ref[...] = x
    xm = _fold_max(x)

    @pl.when(j == 0)
    def _():
        macc[...] = xm

    @pl.when(j > 0)
    def _():
        macc[...] = jnp.maximum(macc[...], xm)

    @pl.when(j == NT - 1)
    def _():
        m_ref[...] = macc[...].max(axis=1, keepdims=True)


def _a1(cond, uncond, mask):
    return pl.pallas_call(
        _a1_body,
        grid=(NT,),
        in_specs=[
            pl.BlockSpec((B, W), lambda j: (0, j)),
            pl.BlockSpec((B, W), lambda j: (0, j)),
            pl.BlockSpec((1, W), lambda j: (0, j)),
        ],
        out_specs=[
            pl.BlockSpec((B, W), lambda j: (0, j)),
            pl.BlockSpec((B, 1), lambda j: (0, 0)),
        ],
        out_shape=[
            jax.ShapeDtypeStruct((B, V), jnp.float32),
            jax.ShapeDtypeStruct((B, 1), jnp.float32),
        ],
        scratch_shapes=[pltpu.VMEM((B, 128), jnp.float32)],
    )(cond, uncond, mask)


def _a2_body(mix_ref, m_ref, mass_ref, cnt_ref, s_ref, massacc, cntacc, sacc):
    j = pl.program_id(0)
    x = mix_ref[...]
    m = m_ref[...]
    col = j * W + lax.broadcasted_iota(jnp.int32, (B, W), 1)
    valid = col < V
    x = jnp.where(valid, x, -jnp.inf)
    e = jnp.exp(x - m)
    colok = valid & (col != 1) & (col != 2)
    e_ok = jnp.where(colok, e, 0.0)

    se = _fold_sum(e)

    @pl.when(j == 0)
    def _():
        sacc[...] = se
        massacc[...] = jnp.zeros_like(massacc)
        cntacc[...] = jnp.zeros_like(cntacc)

    @pl.when(j > 0)
    def _():
        sacc[...] = sacc[...] + se

    for r in range(NR):
        ge = x >= (m - RUNGS[r])
        mr = _fold_sum(jnp.where(ge, e_ok, 0.0))
        massacc[r] = massacc[r] + mr
    for ci, r in enumerate(CNT_RUNGS):
        ge = x >= (m - RUNGS[r])
        cr = _fold_sum(jnp.where(ge & colok, 1.0, 0.0))
        cntacc[ci] = cntacc[ci] + cr

    @pl.when(j == NT - 1)
    def _():
        for r in range(NR):
            mass_ref[:, r:r + 1] = massacc[r].sum(axis=1, keepdims=True)
        for ci in range(len(CNT_RUNGS)):
            cnt_ref[:, ci:ci + 1] = cntacc[ci].sum(axis=1, keepdims=True)
        s_ref[...] = sacc[...].sum(axis=1, keepdims=True)


def _a2(mixed, m):
    return pl.pallas_call(
        _a2_body,
        grid=(NT,),
        in_specs=[
            pl.BlockSpec((B, W), lambda j: (0, j)),
            pl.BlockSpec((B, 1), lambda j: (0, 0)),
        ],
        out_specs=[
            pl.BlockSpec((B, 128), lambda j: (0, 0)),
            pl.BlockSpec((B, 128), lambda j: (0, 0)),
            pl.BlockSpec((B, 1), lambda j: (0, 0)),
        ],
        out_shape=[
            jax.ShapeDtypeStruct((B, 128), jnp.float32),
            jax.ShapeDtypeStruct((B, 128), jnp.float32),
            jax.ShapeDtypeStruct((B, 1), jnp.float32),
        ],
        scratch_shapes=[
            pltpu.VMEM((NR, B, 128), jnp.float32),
            pltpu.VMEM((len(CNT_RUNGS), B, 128), jnp.float32),
            pltpu.VMEM((B, 128), jnp.float32),
        ],
    )(mixed, m)


def _sc_compact_body(mix_hbm, tau_hbm, cv_hbm, ci_hbm, cnt_hbm,
                     rowbuf, vbuf, ibuf, tbuf, cbuf):
    info = plsc.get_sparse_core_info()
    nc = info.num_cores
    wid = lax.axis_index("s") * nc + lax.axis_index("c")
    segv = V // NSEG
    for k in range(B // 32):
        r = wid * (B // 32) + k
        pltpu.sync_copy(mix_hbm.at[r], rowbuf)
        pltpu.sync_copy(tau_hbm.at[r], tbuf)
        tauv = tbuf[...]

        def body_fn(i, offs):
            iv0 = lax.iota(jnp.int32, 16) + i * 16
            new = []
            for s in range(NSEG):
                xv = rowbuf[pl.ds(s * segv + i * 16, 16)]
                ok = xv >= tauv
                if s == 0:
                    idv = iv0
                    ok = ok & ((idv < 1) | (idv > 2))
                else:
                    idv = iv0 + (s * segv)
                ok = ok & (offs[s] < QUOTA - 15)
                base = s * QUOTA
                plsc.store_compressed(vbuf.at[pl.ds(base + offs[s], 16)],
                                      xv, mask=ok)
                plsc.store_compressed(ibuf.at[pl.ds(base + offs[s], 16)],
                                      idv, mask=ok)
                n = plsc.all_reduce_population_count(ok)[0]
                new.append(offs[s] + n)
            return tuple(new)

        offs = lax.fori_loop(0, segv // 16, body_fn,
                             tuple(jnp.int32(0) for _ in range(NSEG)),
                             unroll=4)
        lanes = lax.iota(jnp.int32, 16)
        cvec = jnp.zeros((16,), jnp.int32)
        for s in range(NSEG):
            cvec = jnp.where(lanes == s, offs[s], cvec)
        cbuf[...] = cvec
        pltpu.sync_copy(vbuf, cv_hbm.at[r])
        pltpu.sync_copy(ibuf, ci_hbm.at[r])
        pltpu.sync_copy(cbuf, cnt_hbm.at[r])


def _sc_compact(mixed, tau16):
    mesh = plsc.VectorSubcoreMesh(core_axis_name="c", subcore_axis_name="s")
    f = pl.kernel(
        _sc_compact_body,
        out_type=[
            jax.ShapeDtypeStruct((B, CP), jnp.float32),
            jax.ShapeDtypeStruct((B, CP), jnp.int32),
            jax.ShapeDtypeStruct((B, 16), jnp.int32),
        ],
        mesh=mesh,
        compiler_params=pltpu.CompilerParams(use_tc_tiling_on_sc=False, needs_layout_passes=False),
        scratch_types=[
            pltpu.VMEM((V,), jnp.float32),
            pltpu.VMEM((CP,), jnp.float32),
            pltpu.VMEM((CP,), jnp.int32),
            pltpu.VMEM((16,), jnp.float32),
            pltpu.VMEM((16,), jnp.int32),
        ],
    )
    return f(mixed, tau16)


def _rol(x, sh):
    return jnp.concatenate([x[:, sh:], x[:, :sh]], axis=1)


def _ror(x, sh):
    return jnp.concatenate([x[:, -sh:], x[:, :-sh]], axis=1)


def _c_body(p_ref, i_ref, g_ref, tok_ref, pwin_ref):
    pv = p_ref[...]
    iv = i_ref[...]
    br = pv.shape[0]
    cols = lax.broadcasted_iota(jnp.int32, (br, C), 1)
    k = 2
    while k <= C:
        indesc = (cols & k) == 0
        j = k // 2
        while j >= 1:
            lower = (cols & j) == 0
            pp = jnp.where(lower, _rol(pv, j), _ror(pv, j))
            ip = jnp.where(lower, _rol(iv, j), _ror(iv, j))
            take_max = indesc == lower
            plarger = (pp > pv) | ((pp == pv) & (ip < iv))
            takep = take_max == plarger
            pv = jnp.where(takep, pp, pv)
            iv = jnp.where(takep, ip, iv)
            j //= 2
        k *= 2
    sp = jnp.where(pv >= 0.0, pv, 0.0)
    incl = sp
    sh = 1
    while sh < C:
        z = jnp.zeros((br, sh), jnp.float32)
        incl = incl + jnp.concatenate([z, incl[:, :C - sh]], axis=1)
        sh *= 2
    kept = ((incl - sp) < TOPP) & (pv >= 0.0)
    val = jnp.where(kept, jnp.log(jnp.maximum(pv, 1e-37)) + g_ref[...], -1e30)
    w = jnp.argmax(val, axis=-1).astype(jnp.int32)
    hot = cols == w[:, None]
    tok_ref[...] = jnp.sum(jnp.where(hot, iv, 0), axis=1, keepdims=True)
    pwin_ref[...] = jnp.sum(jnp.where(hot, sp, 0.0), axis=1, keepdims=True)


def _c_sample(p, idxs, g):
    return pl.pallas_call(
        _c_body,
        out_shape=[
            jax.ShapeDtypeStruct((B, 1), jnp.int32),
            jax.ShapeDtypeStruct((B, 1), jnp.float32),
        ],
    )(p, idxs, g)


def _threefry2x32(k1, k2, x0, x1):
    rot = ((13, 15, 26, 6), (17, 29, 16, 24))

    def rotl(x, d):
        return (x << jnp.uint32(d)) | (x >> jnp.uint32(32 - d))

    ks = (k1, k2, k1 ^ k2 ^ jnp.uint32(0x1BD11BDA))
    x0 = x0 + ks[0]
    x1 = x1 + ks[1]
    for i in range(5):
        for r in rot[i % 2]:
            x0 = x0 + x1
            x1 = rotl(x1, r)
            x1 = x1 ^ x0
        x0 = x0 + ks[(i + 1) % 3]
        x1 = x1 + ks[(i + 2) % 3] + jnp.uint32(i + 1)
    return x0, x1


def _partial_gumbel(kd):
    b = lax.broadcasted_iota(jnp.uint32, (B, C), 0)
    j = lax.broadcasted_iota(jnp.uint32, (B, C), 1)
    c_lo = b * jnp.uint32(V) + j
    c_hi = jnp.zeros((B, C), jnp.uint32)
    o1, o2 = _threefry2x32(kd[0], kd[1], c_hi, c_lo)
    bits = o1 ^ o2
    fb = (bits >> jnp.uint32(9)) | jnp.uint32(0x3F800000)
    floats = lax.bitcast_convert_type(fb, jnp.float32) - jnp.float32(1.0)
    tiny = jnp.float32(jnp.finfo(jnp.float32).tiny)
    u = jnp.maximum(tiny, floats * jnp.float32(1.0) + tiny)
    return -jnp.log(-jnp.log(u))


@jax.jit
def kernel(conditioned_logits, unconditioned_logits, non_image_tokens):
    nit = non_image_tokens.astype(jnp.int32)
    mask = _sc_mask(nit)
    mixed, m = _a1(conditioned_logits, unconditioned_logits,
                   mask.reshape(1, V))
    mass_o, cnt_o, s_o = _a2(mixed, m)
    mass = mass_o[:, :NR]
    cnt = cnt_o[:, :NR][:, jnp.asarray(CNT_BOUND)]
    s = s_o[:, 0]
    rungs = jnp.asarray(RUNGS, jnp.float32)
    need = jnp.float32(TOPP) * s * jnp.float32(1.00001)
    ok = (mass >= need[:, None]) & (cnt <= CP)
    fits = cnt <= CP
    fallback = jnp.where(fits.any(axis=1),
                         NR - 1 - jnp.argmax(fits[:, ::-1], axis=1), 0)
    rung_idx = jnp.where(ok.any(axis=1), jnp.argmax(ok, axis=1), fallback)
    tau = m[:, 0] - rungs[rung_idx]
    tau16 = jnp.broadcast_to(tau[:, None], (B, 16))

    cand_v, cand_i, cnt16 = _sc_compact(mixed, tau16)
    pos = jnp.arange(CP, dtype=jnp.int32)
    segidx = pos // QUOTA
    seg_cnt = cnt16[:, segidx]
    validc = (pos % QUOTA)[None, :] < seg_cnt
    logs = jnp.log(s)
    lp = (cand_v - m) - logs[:, None]
    p = jnp.where(validc, jnp.exp(lp), jnp.float32(-1.0))
    idxs = jnp.where(validc, cand_i, jnp.int32(2 ** 30))
    p = jnp.concatenate(
        [p, jnp.full((B, C - CP), -1.0, jnp.float32)], axis=1)
    idxs = jnp.concatenate(
        [idxs, jnp.full((B, C - CP), 2 ** 30, jnp.int32)], axis=1)

    kd = jax.random.key_data(jax.random.fold_in(jax.random.key(0), 123))
    g = _partial_gumbel(kd.astype(jnp.uint32))

    tok, pwin = _c_sample(p, idxs, g)
    next_scores = jnp.log(pwin[:, 0])
    next_toks = tok[:, 0]
    return next_scores, next_toks
